# Initial kernel scaffold; baseline (speedup 1.0000x reference)
import jax
import jax.numpy as jnp
from jax.experimental import pallas as pl


def kernel(x, W, l1_centers, l2_centers, l3_centers):
    raise NotImplementedError("write your pallas kernel here")


# speedup vs baseline: 20.9418x; 20.9418x over previous
import jax
import jax.numpy as jnp
from jax import lax
from jax.experimental import pallas as pl

_TB = 1024


def _route_body(x_ref, wt_ref, c2_ref, c3_ref, out_ref):
    t = x_ref.shape[0]
    pos = jnp.dot(x_ref[...], wt_ref[...], preferred_element_type=jnp.float32)
    px = pos[:, 0:1]
    py = pos[:, 1:2]
    pz = pos[:, 2:3]
    inf = jnp.float32(jnp.inf)

    c2 = c2_ref[...]
    d2 = ((px - c2[0:1, :]) ** 2 + (py - c2[1:2, :]) ** 2
          + (pz - c2[2:3, :]) ** 2)
    it16 = lax.broadcasted_iota(jnp.int32, (t, 16), 1)
    sels = []
    d = d2
    for _ in range(8):
        m = jnp.min(d, axis=1, keepdims=True)
        g = jnp.min(jnp.where(d == m, it16, 64), axis=1, keepdims=True)
        sels.append(g)
        d = jnp.where(it16 == g, inf, d)

    c3 = c3_ref[...]
    d3 = ((px - c3[0:1, :]) ** 2 + (py - c3[1:2, :]) ** 2
          + (pz - c3[2:3, :]) ** 2)
    it64 = lax.broadcasted_iota(jnp.int32, (t, 64), 1)
    grp = it64 >> 2
    allowed = grp == sels[0]
    for i in range(1, 8):
        allowed = allowed | (grp == sels[i])
    d = jnp.where(allowed, d3, inf)
    outs = []
    for _ in range(8):
        m = jnp.min(d, axis=1, keepdims=True)
        e = jnp.min(jnp.where(d == m, it64, 9999), axis=1, keepdims=True)
        outs.append(e)
        d = jnp.where(it64 == e, inf, d)
    out_ref[...] = jnp.concatenate(outs, axis=1)


def kernel(x, W, l1_centers, l2_centers, l3_centers):
    del l1_centers
    b, k = x.shape
    wt = W.T
    c2 = l2_centers.reshape(16, 3).T
    c3 = l3_centers.reshape(64, 3).T
    return pl.pallas_call(
        _route_body,
        grid=(b // _TB,),
        in_specs=[
            pl.BlockSpec((_TB, k), lambda i: (i, 0)),
            pl.BlockSpec((k, 3), lambda i: (0, 0)),
            pl.BlockSpec((3, 16), lambda i: (0, 0)),
            pl.BlockSpec((3, 64), lambda i: (0, 0)),
        ],
        out_specs=pl.BlockSpec((_TB, 8), lambda i: (i, 0)),
        out_shape=jax.ShapeDtypeStruct((b, 8), jnp.int32),
    )(x, wt, c2, c3)


# speedup vs baseline: 37.4725x; 1.7894x over previous
import jax
import jax.numpy as jnp
from jax import lax
from jax.experimental import pallas as pl

_TB = 1024


def _route_body(x_ref, wt_ref, c2_ref, c3_ref, out_ref):
    t = x_ref.shape[0]
    pos = jnp.dot(x_ref[...], wt_ref[...], preferred_element_type=jnp.float32)
    px = pos[:, 0:1]
    py = pos[:, 1:2]
    pz = pos[:, 2:3]
    inf = jnp.float32(jnp.inf)

    c2 = c2_ref[...]
    d2 = ((px - c2[0:1, :]) ** 2 + (py - c2[1:2, :]) ** 2
          + (pz - c2[2:3, :]) ** 2)
    it16 = lax.broadcasted_iota(jnp.int32, (t, 16), 1)
    sels = []
    d = d2
    for _ in range(8):
        g = jnp.argmin(d, axis=1).astype(jnp.int32).reshape(t, 1)
        sels.append(g)
        d = jnp.where(it16 == g, inf, d)

    c3 = c3_ref[...]
    d3 = ((px - c3[0:1, :]) ** 2 + (py - c3[1:2, :]) ** 2
          + (pz - c3[2:3, :]) ** 2)
    it64 = lax.broadcasted_iota(jnp.int32, (t, 64), 1)
    grp = it64 >> 2
    allowed = grp == sels[0]
    for i in range(1, 8):
        allowed = allowed | (grp == sels[i])
    d = jnp.where(allowed, d3, inf)
    outs = []
    for _ in range(8):
        e = jnp.argmin(d, axis=1).astype(jnp.int32).reshape(t, 1)
        outs.append(e)
        d = jnp.where(it64 == e, inf, d)
    out_ref[...] = jnp.concatenate(outs, axis=1)


def kernel(x, W, l1_centers, l2_centers, l3_centers):
    del l1_centers
    b, k = x.shape
    wt = W.T
    c2 = l2_centers.reshape(16, 3).T
    c3 = l3_centers.reshape(64, 3).T
    return pl.pallas_call(
        _route_body,
        grid=(b // _TB,),
        in_specs=[
            pl.BlockSpec((_TB, k), lambda i: (i, 0)),
            pl.BlockSpec((k, 3), lambda i: (0, 0)),
            pl.BlockSpec((3, 16), lambda i: (0, 0)),
            pl.BlockSpec((3, 64), lambda i: (0, 0)),
        ],
        out_specs=pl.BlockSpec((_TB, 8), lambda i: (i, 0)),
        out_shape=jax.ShapeDtypeStruct((b, 8), jnp.int32),
    )(x, wt, c2, c3)
